# Initial kernel scaffold; baseline (speedup 1.0000x reference)
import jax
import jax.numpy as jnp
from jax.experimental import pallas as pl


def kernel(x, edge_index, W1, b1, W2, b2, W3, b3, W4, b4, W5, b5):
    raise NotImplementedError("write your pallas kernel here")


# speedup vs baseline: 5.7760x; 5.7760x over previous
import functools

import jax
import jax.numpy as jnp
from jax import lax
from jax.experimental import pallas as pl
from jax.experimental.pallas import tpu as pltpu
from jax.experimental.pallas import tpu_sc as plsc

_NC = 2
_NS = 16
_EB = 1536
_RB = 1024


def _mesh():
  return plsc.VectorSubcoreMesh(
      core_axis_name="c", subcore_axis_name="s",
      num_cores=_NC, num_subcores=_NS)


def _fill2(ref, val):
  v = jnp.full((16,), val, ref.dtype)

  def body(i, _):
    ref[i] = v
    return 0

  lax.fori_loop(0, ref.shape[0], body, 0)


def _fill1(ref, val):
  v = jnp.full((16,), val, ref.dtype)

  def body(i, _):
    ref[pl.ds(pl.multiple_of(i * 16, 16), 16)] = v
    return 0

  lax.fori_loop(0, ref.shape[0] // 16, body, 0)


def _zero_slice2(agg_s, zero_v, row0, rpt):
  nz = rpt // _EB
  rem = rpt - nz * _EB
  for j in range(nz):
    pltpu.sync_copy(zero_v, agg_s.at[pl.ds(row0 + j * _EB, _EB)])
  if rem:
    pltpu.sync_copy(zero_v.at[pl.ds(0, rem)],
                    agg_s.at[pl.ds(row0 + nz * _EB, rem)])


@functools.cache
def _deg_kernel(EP, NP):
  ept = EP // _NS
  nblk = ept // _EB
  rpt = NP // _NS

  @functools.partial(
      pl.kernel,
      out_type=jax.ShapeDtypeStruct((_NC * NP,), jnp.float32),
      mesh=_mesh(),
      compiler_params=pltpu.CompilerParams(use_tc_tiling_on_sc=False),
      scratch_types=[
          pltpu.VMEM((_EB,), jnp.int32),
          pltpu.VMEM((_EB,), jnp.float32),
          pltpu.VMEM((rpt,), jnp.float32),
          pltpu.VMEM_SHARED((NP,), jnp.float32),
      ],
  )
  def k(ei_hbm, out_hbm, idx_v, ones_v, zb_v, deg_s):
    c = lax.axis_index("c")
    s = lax.axis_index("s")
    _fill1(ones_v, 1.0)
    _fill1(zb_v, 0.0)
    row0 = pl.multiple_of(s * rpt, 128)
    pltpu.sync_copy(zb_v, deg_s.at[pl.ds(row0, rpt)])
    plsc.subcore_barrier()

    def blk(i, _):
      e0 = pl.multiple_of(c * EP + s * ept + i * _EB, 128)
      pltpu.sync_copy(ei_hbm.at[pl.ds(e0, _EB)], idx_v)
      pltpu.sync_copy(ones_v, deg_s.at[idx_v], add=True)
      return 0

    lax.fori_loop(0, nblk, blk, 0)
    plsc.subcore_barrier()
    out0 = pl.multiple_of(c * NP + row0, 128)
    pltpu.sync_copy(deg_s.at[pl.ds(row0, rpt)], out_hbm.at[pl.ds(out0, rpt)])

  return k


@functools.cache
def _agg16_kernel(EP, NP):
  ept = EP // (_NC * _NS)
  nblk = ept // _EB
  rpt = NP // _NS

  @functools.partial(
      pl.kernel,
      out_type=jax.ShapeDtypeStruct((_NC, NP, 16), jnp.float32),
      mesh=_mesh(),
      compiler_params=pltpu.CompilerParams(use_tc_tiling_on_sc=False),
      scratch_types=[
          pltpu.VMEM((_EB,), jnp.int32),
          pltpu.VMEM((_EB,), jnp.int32),
          pltpu.VMEM((_EB, 16), jnp.float32),
          pltpu.VMEM_SHARED((NP, 16), jnp.float32),
      ],
  )
  def k(tbl_hbm, src_hbm, dst_hbm, out_hbm, src_v, dst_v, rows_v, agg_s):
    c = lax.axis_index("c")
    s = lax.axis_index("s")
    _fill2(rows_v, 0.0)
    row0 = pl.multiple_of(s * rpt, 8)
    _zero_slice2(agg_s, rows_v, row0, rpt)
    plsc.subcore_barrier()

    def blk(i, _):
      e0 = pl.multiple_of(c * (EP // 2) + s * ept + i * _EB, 128)
      pltpu.sync_copy(src_hbm.at[pl.ds(e0, _EB)], src_v)
      pltpu.sync_copy(dst_hbm.at[pl.ds(e0, _EB)], dst_v)
      pltpu.sync_copy(tbl_hbm.at[src_v], rows_v)
      pltpu.sync_copy(rows_v, agg_s.at[dst_v], add=True)
      return 0

    lax.fori_loop(0, nblk, blk, 0)
    plsc.subcore_barrier()
    pltpu.sync_copy(agg_s.at[pl.ds(row0, rpt)], out_hbm.at[c, pl.ds(row0, rpt)])

  return k


@functools.cache
def _agg128_kernel(EP, NP):
  ept = EP // _NS
  nblk = ept // _EB
  rpt = NP // _NS
  nch = 8 // _NC

  @functools.partial(
      pl.kernel,
      out_type=jax.ShapeDtypeStruct((8, NP, 16), jnp.float32),
      mesh=_mesh(),
      compiler_params=pltpu.CompilerParams(use_tc_tiling_on_sc=False),
      scratch_types=[
          pltpu.VMEM((_EB,), jnp.int32),
          pltpu.VMEM((_EB,), jnp.int32),
          pltpu.VMEM((_EB, 16), jnp.float32),
          pltpu.VMEM_SHARED((NP, 16), jnp.float32),
      ],
  )
  def k(h_hbm, src_hbm, dst_hbm, out_hbm, src_v, dst_v, rows_v, agg_s):
    c = lax.axis_index("c")
    s = lax.axis_index("s")
    row0 = pl.multiple_of(s * rpt, 8)
    for cc in range(nch):
      ch = c * nch + cc
      _fill2(rows_v, 0.0)
      _zero_slice2(agg_s, rows_v, row0, rpt)
      plsc.subcore_barrier()

      def blk(i, _):
        e0 = pl.multiple_of(s * ept + i * _EB, 128)
        pltpu.sync_copy(src_hbm.at[pl.ds(e0, _EB)], src_v)
        pltpu.sync_copy(dst_hbm.at[pl.ds(e0, _EB)], dst_v)
        pltpu.sync_copy(h_hbm.at[ch].at[src_v], rows_v)
        pltpu.sync_copy(rows_v, agg_s.at[dst_v], add=True)
        return 0

      lax.fori_loop(0, nblk, blk, 0)
      plsc.subcore_barrier()
      pltpu.sync_copy(agg_s.at[pl.ds(row0, rpt)],
                      out_hbm.at[ch, pl.ds(row0, rpt)])

  return k


def _norm(deg_blk):
  return lax.rsqrt(jnp.maximum(deg_blk, 1.0))


def _leaky(z):
  return jnp.where(z >= 0, z, 0.01 * z)


def _cat8(a_ref):
  return jnp.concatenate([a_ref[c] for c in range(8)], axis=1)


def _split8(o_ref, h):
  for c in range(8):
    o_ref[c] = h[:, c * 16:(c + 1) * 16]


def _scale_x_tc(xp, dego):
  NP = xp.shape[0]

  def body(x_ref, d_ref, o_ref):
    o_ref[...] = x_ref[...] * _norm(d_ref[...])

  return pl.pallas_call(
      body,
      grid=(NP // _RB,),
      in_specs=[
          pl.BlockSpec((_RB, 16), lambda i: (i, 0)),
          pl.BlockSpec((_RB, 1), lambda i: (i, 0)),
      ],
      out_specs=pl.BlockSpec((_RB, 16), lambda i: (i, 0)),
      out_shape=jax.ShapeDtypeStruct((NP, 16), jnp.float32),
  )(xp, dego)


def _layer1_tc(p, degi, dego, W1p, b1):
  NP = p.shape[1]

  def body(p_ref, di_ref, do_ref, w_ref, b_ref, o_ref):
    agg = (p_ref[0] + p_ref[1]) * _norm(di_ref[...])
    z = jnp.dot(agg, w_ref[...], preferred_element_type=jnp.float32)
    z = z + b_ref[...]
    _split8(o_ref, _leaky(z) * _norm(do_ref[...]))

  return pl.pallas_call(
      body,
      grid=(NP // _RB,),
      in_specs=[
          pl.BlockSpec((2, _RB, 16), lambda i: (0, i, 0)),
          pl.BlockSpec((_RB, 1), lambda i: (i, 0)),
          pl.BlockSpec((_RB, 1), lambda i: (i, 0)),
          pl.BlockSpec((16, 128), lambda i: (0, 0)),
          pl.BlockSpec((1, 128), lambda i: (0, 0)),
      ],
      out_specs=pl.BlockSpec((8, _RB, 16), lambda i: (0, i, 0)),
      out_shape=jax.ShapeDtypeStruct((8, NP, 16), jnp.float32),
  )(p, degi, dego, W1p, b1)


def _mid_tc(agg8, degi, dego, W, b):
  NP = agg8.shape[1]

  def body(a_ref, di_ref, do_ref, w_ref, b_ref, o_ref):
    a = _cat8(a_ref) * _norm(di_ref[...])
    z = jnp.dot(a, w_ref[...], preferred_element_type=jnp.float32)
    z = z + b_ref[...]
    _split8(o_ref, _leaky(z) * _norm(do_ref[...]))

  return pl.pallas_call(
      body,
      grid=(NP // _RB,),
      in_specs=[
          pl.BlockSpec((8, _RB, 16), lambda i: (0, i, 0)),
          pl.BlockSpec((_RB, 1), lambda i: (i, 0)),
          pl.BlockSpec((_RB, 1), lambda i: (i, 0)),
          pl.BlockSpec((128, 128), lambda i: (0, 0)),
          pl.BlockSpec((1, 128), lambda i: (0, 0)),
      ],
      out_specs=pl.BlockSpec((8, _RB, 16), lambda i: (0, i, 0)),
      out_shape=jax.ShapeDtypeStruct((8, NP, 16), jnp.float32),
  )(agg8, degi, dego, W, b)


def _layer4_tc(agg8, degi, dego, W4, b4, W5p):
  NP = agg8.shape[1]

  def body(a_ref, di_ref, do_ref, w4_ref, b4_ref, w5_ref, o_ref):
    a = _cat8(a_ref) * _norm(di_ref[...])
    z = jnp.dot(a, w4_ref[...], preferred_element_type=jnp.float32)
    z = z + b4_ref[...]
    h4 = _leaky(z) * _norm(do_ref[...])
    o_ref[...] = jnp.dot(h4, w5_ref[...], preferred_element_type=jnp.float32)

  return pl.pallas_call(
      body,
      grid=(NP // _RB,),
      in_specs=[
          pl.BlockSpec((8, _RB, 16), lambda i: (0, i, 0)),
          pl.BlockSpec((_RB, 1), lambda i: (i, 0)),
          pl.BlockSpec((_RB, 1), lambda i: (i, 0)),
          pl.BlockSpec((128, 128), lambda i: (0, 0)),
          pl.BlockSpec((1, 128), lambda i: (0, 0)),
          pl.BlockSpec((128, 16), lambda i: (0, 0)),
      ],
      out_specs=pl.BlockSpec((_RB, 16), lambda i: (i, 0)),
      out_shape=jax.ShapeDtypeStruct((NP, 16), jnp.float32),
  )(agg8, degi, dego, W4, b4, W5p)


def _final_tc(p, degi, b5p):
  NP = p.shape[1]

  def body(p_ref, di_ref, b_ref, o_ref):
    o_ref[...] = (p_ref[0] + p_ref[1]) * _norm(di_ref[...]) + b_ref[...]

  return pl.pallas_call(
      body,
      grid=(NP // _RB,),
      in_specs=[
          pl.BlockSpec((2, _RB, 16), lambda i: (0, i, 0)),
          pl.BlockSpec((_RB, 1), lambda i: (i, 0)),
          pl.BlockSpec((1, 16), lambda i: (0, 0)),
      ],
      out_specs=pl.BlockSpec((_RB, 16), lambda i: (i, 0)),
      out_shape=jax.ShapeDtypeStruct((NP, 16), jnp.float32),
  )(p, degi, b5p)


def kernel(x, edge_index, W1, b1, W2, b2, W3, b3, W4, b4, W5, b5):
  N, Fin = x.shape
  E = edge_index.shape[1]
  NP = ((N + _RB - 1) // _RB) * _RB
  eb_all = _NC * _NS * _EB
  EP = ((E + eb_all - 1) // eb_all) * eb_all
  assert NP % (_NS * 128) == 0

  npad = EP - E
  fill = N + (jnp.arange(npad, dtype=jnp.int32) % (NP - N))
  srcp = jnp.concatenate([edge_index[0], fill])
  dstp = jnp.concatenate([edge_index[1], fill])
  ei_flat = jnp.concatenate([srcp, dstp])

  deg = _deg_kernel(EP, NP)(ei_flat).reshape(_NC, NP)
  dego = deg[0].reshape(NP, 1)
  degi = deg[1].reshape(NP, 1)

  xp = jnp.pad(x, ((0, NP - N), (0, 16 - Fin)))
  xs = _scale_x_tc(xp, dego)
  p = _agg16_kernel(EP, NP)(xs, srcp, dstp)

  W1p = jnp.pad(W1, ((0, 16 - Fin), (0, 0)))
  h8 = _layer1_tc(p, degi, dego, W1p, b1.reshape(1, 128))

  for W, b in ((W2, b2), (W3, b3)):
    a8 = _agg128_kernel(EP, NP)(h8, srcp, dstp)
    h8 = _mid_tc(a8, degi, dego, W, b.reshape(1, 128))

  a8 = _agg128_kernel(EP, NP)(h8, srcp, dstp)
  W5p = jnp.pad(W5, ((0, 0), (0, 16 - W5.shape[1])))
  g = _layer4_tc(a8, degi, dego, W4, b4.reshape(1, 128), W5p)

  p = _agg16_kernel(EP, NP)(g, srcp, dstp)
  b5p = jnp.pad(b5, (0, 16 - b5.shape[0])).reshape(1, 16)
  out16 = _final_tc(p, degi, b5p)
  return out16[:N, :3]


# speedup vs baseline: 10.3389x; 1.7900x over previous
import functools

import jax
import jax.numpy as jnp
from jax import lax
from jax.experimental import pallas as pl
from jax.experimental.pallas import tpu as pltpu
from jax.experimental.pallas import tpu_sc as plsc

_NC = 2
_NS = 16
_EB = 1536
_RB = 1024


def _mesh():
  return plsc.VectorSubcoreMesh(
      core_axis_name="c", subcore_axis_name="s",
      num_cores=_NC, num_subcores=_NS)


def _fill2(ref, val):
  v = jnp.full((16,), val, ref.dtype)

  def body(i, _):
    ref[i] = v
    return 0

  lax.fori_loop(0, ref.shape[0], body, 0)


def _fill1(ref, val):
  v = jnp.full((16,), val, ref.dtype)

  def body(i, _):
    ref[pl.ds(pl.multiple_of(i * 16, 16), 16)] = v
    return 0

  lax.fori_loop(0, ref.shape[0] // 16, body, 0)


def _zero_slice2(agg_s, zero_v, row0, rpt):
  nz = rpt // _EB
  rem = rpt - nz * _EB
  for j in range(nz):
    pltpu.sync_copy(zero_v, agg_s.at[pl.ds(row0 + j * _EB, _EB)])
  if rem:
    pltpu.sync_copy(zero_v.at[pl.ds(0, rem)],
                    agg_s.at[pl.ds(row0 + nz * _EB, rem)])


@functools.cache
def _deg_kernel(EP, NP):
  ept = EP // _NS
  nblk = ept // _EB
  rpt = NP // _NS

  @functools.partial(
      pl.kernel,
      out_type=jax.ShapeDtypeStruct((_NC * NP,), jnp.float32),
      mesh=_mesh(),
      compiler_params=pltpu.CompilerParams(use_tc_tiling_on_sc=False),
      scratch_types=[
          pltpu.VMEM((_EB,), jnp.int32),
          pltpu.VMEM((_EB,), jnp.float32),
          pltpu.VMEM((rpt,), jnp.float32),
          pltpu.VMEM_SHARED((NP,), jnp.float32),
      ],
  )
  def k(ei_hbm, out_hbm, idx_v, ones_v, zb_v, deg_s):
    c = lax.axis_index("c")
    s = lax.axis_index("s")
    _fill1(ones_v, 1.0)
    _fill1(zb_v, 0.0)
    row0 = pl.multiple_of(s * rpt, 128)
    pltpu.sync_copy(zb_v, deg_s.at[pl.ds(row0, rpt)])
    plsc.subcore_barrier()

    def blk(i, _):
      e0 = pl.multiple_of(c * EP + s * ept + i * _EB, 128)
      pltpu.sync_copy(ei_hbm.at[pl.ds(e0, _EB)], idx_v)
      pltpu.sync_copy(ones_v, deg_s.at[idx_v], add=True)
      return 0

    lax.fori_loop(0, nblk, blk, 0)
    plsc.subcore_barrier()
    out0 = pl.multiple_of(c * NP + row0, 128)
    pltpu.sync_copy(deg_s.at[pl.ds(row0, rpt)], out_hbm.at[pl.ds(out0, rpt)])

  return k


@functools.cache
def _agg16_kernel(EP, NP):
  ept = EP // (_NC * _NS)
  nblk = ept // _EB
  rpt = NP // _NS

  @functools.partial(
      pl.kernel,
      out_type=jax.ShapeDtypeStruct((_NC, NP, 16), jnp.float32),
      mesh=_mesh(),
      compiler_params=pltpu.CompilerParams(use_tc_tiling_on_sc=False),
      scratch_types=[
          pltpu.VMEM((_EB,), jnp.int32),
          pltpu.VMEM((_EB,), jnp.int32),
          pltpu.VMEM((_EB, 16), jnp.float32),
          pltpu.VMEM_SHARED((NP, 16), jnp.float32),
      ],
  )
  def k(tbl_hbm, ei_hbm, out_hbm, src_v, dst_v, rows_v, agg_s):
    c = lax.axis_index("c")
    s = lax.axis_index("s")
    _fill2(rows_v, 0.0)
    row0 = pl.multiple_of(s * rpt, 8)
    _zero_slice2(agg_s, rows_v, row0, rpt)
    plsc.subcore_barrier()
    base = pl.multiple_of(c * (EP // 2) + s * ept, 128)

    def blk(i, _):
      e0 = pl.multiple_of(base + i * _EB, 128)
      pltpu.sync_copy(ei_hbm.at[pl.ds(e0, _EB)], src_v)
      pltpu.sync_copy(ei_hbm.at[pl.ds(EP + e0, _EB)], dst_v)
      pltpu.sync_copy(tbl_hbm.at[src_v], rows_v)
      pltpu.sync_copy(rows_v, agg_s.at[dst_v], add=True)
      return 0

    lax.fori_loop(0, nblk, blk, 0)
    plsc.subcore_barrier()
    pltpu.sync_copy(agg_s.at[pl.ds(row0, rpt)], out_hbm.at[c, pl.ds(row0, rpt)])

  return k


@functools.cache
def _agg128_kernel(EP, NP):
  ept = EP // _NS
  nblk = ept // _EB
  rpt = NP // _NS
  nch = 4 // _NC

  @functools.partial(
      pl.kernel,
      out_type=jax.ShapeDtypeStruct((NP, 128), jnp.bfloat16),
      mesh=_mesh(),
      compiler_params=pltpu.CompilerParams(use_tc_tiling_on_sc=False),
      scratch_types=[
          pltpu.VMEM((_EB,), jnp.int32),
          pltpu.VMEM((_EB,), jnp.int32),
          pltpu.VMEM((_EB, 32), jnp.bfloat16),
          pltpu.VMEM_SHARED((NP, 32), jnp.bfloat16),
      ],
  )
  def k(h_hbm, idx8_hbm, ei_hbm, z_hbm, out_hbm, idx_v, dst_v, rows_v, agg_s):
    c = lax.axis_index("c")
    s = lax.axis_index("s")
    row0 = pl.multiple_of(s * rpt, 8)
    base = pl.multiple_of(s * ept, 128)
    for cc in range(nch):
      ch = c * nch + cc
      colo = pl.multiple_of(ch * 32, 32)
      pltpu.sync_copy(z_hbm.at[pl.ds(row0, rpt)], agg_s.at[pl.ds(row0, rpt)])
      plsc.subcore_barrier()

      def blk(i, _):
        e0 = pl.multiple_of(base + i * _EB, 128)
        pltpu.sync_copy(idx8_hbm.at[pl.ds(ch * EP + e0, _EB)], idx_v)
        pltpu.sync_copy(ei_hbm.at[pl.ds(EP + e0, _EB)], dst_v)
        pltpu.sync_copy(h_hbm.at[idx_v], rows_v)
        pltpu.sync_copy(rows_v, agg_s.at[dst_v], add=True)
        return 0

      lax.fori_loop(0, nblk, blk, 0)
      plsc.subcore_barrier()
      pltpu.sync_copy(agg_s.at[pl.ds(row0, rpt)],
                      out_hbm.at[pl.ds(row0, rpt), pl.ds(colo, 32)])

  return k


def _norm(deg_blk):
  return lax.rsqrt(jnp.maximum(deg_blk, 1.0))


def _leaky(z):
  return jnp.where(z >= 0, z, 0.01 * z)


def _scale_x_tc(xp, dego):
  NP = xp.shape[0]

  def body(x_ref, d_ref, o_ref):
    o_ref[...] = x_ref[...] * _norm(d_ref[...])

  return pl.pallas_call(
      body,
      grid=(NP // _RB,),
      in_specs=[
          pl.BlockSpec((_RB, 16), lambda i: (i, 0)),
          pl.BlockSpec((_RB, 1), lambda i: (i, 0)),
      ],
      out_specs=pl.BlockSpec((_RB, 16), lambda i: (i, 0)),
      out_shape=jax.ShapeDtypeStruct((NP, 16), jnp.float32),
  )(xp, dego)


def _layer1_tc(p, degi, dego, W1p, b1):
  NP = p.shape[1]

  def body(p_ref, di_ref, do_ref, w_ref, b_ref, o_ref):
    agg = (p_ref[0] + p_ref[1]) * _norm(di_ref[...])
    z = jnp.dot(agg, w_ref[...], preferred_element_type=jnp.float32)
    z = z + b_ref[...]
    o_ref[...] = (_leaky(z) * _norm(do_ref[...])).astype(jnp.bfloat16)

  return pl.pallas_call(
      body,
      grid=(NP // _RB,),
      in_specs=[
          pl.BlockSpec((2, _RB, 16), lambda i: (0, i, 0)),
          pl.BlockSpec((_RB, 1), lambda i: (i, 0)),
          pl.BlockSpec((_RB, 1), lambda i: (i, 0)),
          pl.BlockSpec((16, 128), lambda i: (0, 0)),
          pl.BlockSpec((1, 128), lambda i: (0, 0)),
      ],
      out_specs=pl.BlockSpec((_RB, 128), lambda i: (i, 0)),
      out_shape=jax.ShapeDtypeStruct((NP, 128), jnp.bfloat16),
  )(p, degi, dego, W1p, b1)


def _mid_tc(agg, degi, dego, W, b):
  NP = agg.shape[0]

  def body(a_ref, di_ref, do_ref, w_ref, b_ref, o_ref):
    a = a_ref[...].astype(jnp.float32) * _norm(di_ref[...])
    z = jnp.dot(a, w_ref[...], preferred_element_type=jnp.float32)
    z = z + b_ref[...]
    o_ref[...] = (_leaky(z) * _norm(do_ref[...])).astype(jnp.bfloat16)

  return pl.pallas_call(
      body,
      grid=(NP // _RB,),
      in_specs=[
          pl.BlockSpec((_RB, 128), lambda i: (i, 0)),
          pl.BlockSpec((_RB, 1), lambda i: (i, 0)),
          pl.BlockSpec((_RB, 1), lambda i: (i, 0)),
          pl.BlockSpec((128, 128), lambda i: (0, 0)),
          pl.BlockSpec((1, 128), lambda i: (0, 0)),
      ],
      out_specs=pl.BlockSpec((_RB, 128), lambda i: (i, 0)),
      out_shape=jax.ShapeDtypeStruct((NP, 128), jnp.bfloat16),
  )(agg, degi, dego, W, b)


def _layer4_tc(agg, degi, dego, W4, b4, W5p):
  NP = agg.shape[0]

  def body(a_ref, di_ref, do_ref, w4_ref, b4_ref, w5_ref, o_ref):
    a = a_ref[...].astype(jnp.float32) * _norm(di_ref[...])
    z = jnp.dot(a, w4_ref[...], preferred_element_type=jnp.float32)
    z = z + b4_ref[...]
    h4 = _leaky(z) * _norm(do_ref[...])
    o_ref[...] = jnp.dot(h4, w5_ref[...], preferred_element_type=jnp.float32)

  return pl.pallas_call(
      body,
      grid=(NP // _RB,),
      in_specs=[
          pl.BlockSpec((_RB, 128), lambda i: (i, 0)),
          pl.BlockSpec((_RB, 1), lambda i: (i, 0)),
          pl.BlockSpec((_RB, 1), lambda i: (i, 0)),
          pl.BlockSpec((128, 128), lambda i: (0, 0)),
          pl.BlockSpec((1, 128), lambda i: (0, 0)),
          pl.BlockSpec((128, 16), lambda i: (0, 0)),
      ],
      out_specs=pl.BlockSpec((_RB, 16), lambda i: (i, 0)),
      out_shape=jax.ShapeDtypeStruct((NP, 16), jnp.float32),
  )(agg, degi, dego, W4, b4, W5p)


def _final_tc(p, degi, b5p):
  NP = p.shape[1]

  def body(p_ref, di_ref, b_ref, o_ref):
    o_ref[...] = (p_ref[0] + p_ref[1]) * _norm(di_ref[...]) + b_ref[...]

  return pl.pallas_call(
      body,
      grid=(NP // _RB,),
      in_specs=[
          pl.BlockSpec((2, _RB, 16), lambda i: (0, i, 0)),
          pl.BlockSpec((_RB, 1), lambda i: (i, 0)),
          pl.BlockSpec((1, 16), lambda i: (0, 0)),
      ],
      out_specs=pl.BlockSpec((_RB, 16), lambda i: (i, 0)),
      out_shape=jax.ShapeDtypeStruct((NP, 16), jnp.float32),
  )(p, degi, b5p)


def kernel(x, edge_index, W1, b1, W2, b2, W3, b3, W4, b4, W5, b5):
  N, Fin = x.shape
  E = edge_index.shape[1]
  NP = ((N + _RB - 1) // _RB) * _RB
  eb_all = _NC * _NS * _EB
  EP = ((E + eb_all - 1) // eb_all) * eb_all
  assert NP % (_NS * 128) == 0

  npad = EP - E
  fill = N + (jnp.arange(npad, dtype=jnp.int32) % (NP - N))
  ei_flat = jnp.concatenate(
      [edge_index[0], fill, edge_index[1], fill])
  src4 = jnp.concatenate([edge_index[0], fill]) * 4
  idx4 = jnp.concatenate([src4 + cc for cc in range(4)])

  deg = _deg_kernel(EP, NP)(ei_flat).reshape(_NC, NP)
  dego = deg[0].reshape(NP, 1)
  degi = deg[1].reshape(NP, 1)

  xp = jnp.pad(x, ((0, NP - N), (0, 16 - Fin)))
  xs = _scale_x_tc(xp, dego)
  p = _agg16_kernel(EP, NP)(xs, ei_flat)

  W1p = jnp.pad(W1, ((0, 16 - Fin), (0, 0)))
  h = _layer1_tc(p, degi, dego, W1p, b1.reshape(1, 128))

  z32 = jnp.zeros((NP, 32), jnp.bfloat16)
  for W, b in ((W2, b2), (W3, b3)):
    a = _agg128_kernel(EP, NP)(h.reshape(NP * 4, 32), idx4, ei_flat, z32)
    h = _mid_tc(a, degi, dego, W, b.reshape(1, 128))

  a = _agg128_kernel(EP, NP)(h.reshape(NP * 4, 32), idx4, ei_flat, z32)
  W5p = jnp.pad(W5, ((0, 0), (0, 16 - W5.shape[1])))
  g = _layer4_tc(a, degi, dego, W4, b4.reshape(1, 128), W5p)

  p = _agg16_kernel(EP, NP)(g, ei_flat)
  b5p = jnp.pad(b5, (0, 16 - b5.shape[0])).reshape(1, 16)
  out16 = _final_tc(p, degi, b5p)
  return out16[:N, :3]
